# Initial kernel scaffold; baseline (speedup 1.0000x reference)
import jax
import jax.numpy as jnp
from jax.experimental import pallas as pl


def kernel(x, W):
    raise NotImplementedError("write your pallas kernel here")


# speedup vs baseline: 2.0364x; 2.0364x over previous
import functools
import math

import jax
import jax.numpy as jnp
from jax import lax
from jax.experimental import pallas as pl
from jax.experimental.pallas import tpu as pltpu
from jax.experimental.pallas import tpu_sc as plsc

NC = 2
NS = 16
NW = NC * NS
LANES = 16
CHUNK = 100


def _pos_table(L, D):
    def body(o_ref):
        j = lax.broadcasted_iota(jnp.int32, (L, D), 1)
        pos = lax.broadcasted_iota(jnp.int32, (L, D), 0).astype(jnp.float32)
        k = (j // 2).astype(jnp.float32)
        div = jnp.exp(k * (-2.0 * math.log(10000.0) / D))
        ang = pos * div
        o_ref[...] = jnp.where(j % 2 == 0, jnp.sin(ang), jnp.cos(ang))

    return pl.pallas_call(
        body, out_shape=jax.ShapeDtypeStruct((L, D), jnp.float32)
    )()


@jax.jit
def kernel(x, W):
    B, L = x.shape
    V, D = W.shape
    N = B * L
    assert N % NW == 0
    PW = N // NW
    assert PW % CHUNK == 0
    NCH = PW // CHUNK
    assert L == 2 * CHUNK
    assert PW % L == 0
    DV = D // LANES

    pos = _pos_table(L, D).reshape(2, CHUNK, D)
    x3 = x.astype(jnp.int32).reshape(NW, NCH, CHUNK)

    @functools.partial(
        pl.kernel,
        out_type=jax.ShapeDtypeStruct((N, D), jnp.float32),
        mesh=plsc.VectorSubcoreMesh(core_axis_name="c", subcore_axis_name="s"),
        scratch_types=[
            pltpu.VMEM((NCH, CHUNK), jnp.int32),
            pltpu.VMEM((2, CHUNK, D), jnp.float32),
            pltpu.VMEM((2, CHUNK, D), jnp.float32),
            pltpu.SemaphoreType.DMA,
        ],
        compiler_params=pltpu.CompilerParams(use_tc_tiling_on_sc=False),
    )
    def sc_embed(x_hbm, pos_hbm, W_hbm, out_hbm, idx_v, pos_v, gbuf, gsem):
        wid = lax.axis_index("s") * NC + lax.axis_index("c")
        base = wid * PW
        pltpu.sync_copy(x_hbm.at[wid], idx_v)
        pltpu.sync_copy(pos_hbm, pos_v)

        def do_chunk(ch, ph):
            pltpu.async_copy(W_hbm.at[idx_v.at[ch]], gbuf.at[ph], gsem).wait()

            def row(j, carry):
                for kcol in range(DV):
                    sl = pl.ds(kcol * LANES, LANES)
                    gbuf[ph, j, sl] = gbuf[ph, j, sl] + pos_v[ph, j, sl]
                return carry

            lax.fori_loop(0, CHUNK, row, 0, unroll=2)
            pltpu.sync_copy(
                gbuf.at[ph], out_hbm.at[pl.ds(base + ch * CHUNK, CHUNK)]
            )

        def pair(p, carry):
            do_chunk(2 * p, 0)
            do_chunk(2 * p + 1, 1)
            return carry

        lax.fori_loop(0, NCH // 2, pair, 0)

    out = sc_embed(x3, pos, W)
    return out.reshape(B, L, D)


# speedup vs baseline: 4.4705x; 2.1953x over previous
import functools
import math

import jax
import jax.numpy as jnp
from jax import lax
from jax.experimental import pallas as pl
from jax.experimental.pallas import tpu as pltpu
from jax.experimental.pallas import tpu_sc as plsc

NC = 2
NS = 16
NW = NC * NS
LANES = 16
CHUNK = 100
NBUF = 4
AHEAD = 2


def _pos_table(L, D):
    def body(o_ref):
        j = lax.broadcasted_iota(jnp.int32, (L, D), 1)
        pos = lax.broadcasted_iota(jnp.int32, (L, D), 0).astype(jnp.float32)
        k = (j // 2).astype(jnp.float32)
        div = jnp.exp(k * (-2.0 * math.log(10000.0) / D))
        ang = pos * div
        o_ref[...] = jnp.where(j % 2 == 0, jnp.sin(ang), jnp.cos(ang))

    return pl.pallas_call(
        body, out_shape=jax.ShapeDtypeStruct((L, D), jnp.float32)
    )()


@jax.jit
def kernel(x, W):
    B, L = x.shape
    V, D = W.shape
    N = B * L
    assert N % NW == 0
    PW = N // NW
    assert PW % CHUNK == 0
    NCH = PW // CHUNK
    assert L == 2 * CHUNK
    assert PW % L == 0
    assert NCH % NBUF == 0 and NCH // NBUF >= 3
    DV = D // LANES

    pos = _pos_table(L, D).reshape(2, CHUNK, D)
    x3 = x.astype(jnp.int32).reshape(NW, NCH, CHUNK)

    @functools.partial(
        pl.kernel,
        out_type=jax.ShapeDtypeStruct((N, D), jnp.float32),
        mesh=plsc.VectorSubcoreMesh(core_axis_name="c", subcore_axis_name="s"),
        scratch_types=[
            pltpu.VMEM((NCH, CHUNK), jnp.int32),
            pltpu.VMEM((2, CHUNK, D), jnp.float32),
            pltpu.VMEM((NBUF, CHUNK, D), jnp.float32),
        ]
        + [pltpu.SemaphoreType.DMA] * (2 * NBUF),
        compiler_params=pltpu.CompilerParams(use_tc_tiling_on_sc=False),
    )
    def sc_embed(x_hbm, pos_hbm, W_hbm, out_hbm, idx_v, pos_v, gbuf, *sems):
        gsem = sems[:NBUF]
        ssem = sems[NBUF:]
        wid = lax.axis_index("s") * NC + lax.axis_index("c")
        base = wid * PW
        pltpu.sync_copy(x_hbm.at[wid], idx_v)
        pltpu.sync_copy(pos_hbm, pos_v)

        def gather(ch, b):
            return pltpu.make_async_copy(
                W_hbm.at[idx_v.at[ch]], gbuf.at[b], gsem[b]
            )

        def scatter(ch, b):
            return pltpu.make_async_copy(
                gbuf.at[b], out_hbm.at[pl.ds(base + ch * CHUNK, CHUNK)], ssem[b]
            )

        def add_pos(b, half):
            def row(j, carry):
                for kcol in range(DV):
                    sl = pl.ds(kcol * LANES, LANES)
                    gbuf[b, j, sl] = gbuf[b, j, sl] + pos_v[half, j, sl]
                return carry

            lax.fori_loop(0, CHUNK, row, 0, unroll=2)

        def group(p, first, last):
            for i in range(NBUF):
                c = NBUF * p + i
                gather(c, i).wait()
                add_pos(i, i % 2)
                scatter(c, i).start()
                ca = c + AHEAD
                bb = (i + AHEAD) % NBUF
                if not (last and i >= NBUF - AHEAD):
                    if not (first and i < AHEAD):
                        scatter(ca - NBUF, bb).wait()
                    gather(ca, bb).start()

        for c in range(AHEAD):
            gather(c, c).start()
        group(0, True, False)
        lax.fori_loop(1, NCH // NBUF - 1, lambda p, _: (group(p, False, False), 0)[1], 0)
        group(NCH // NBUF - 1, False, True)
        for i in range(NBUF):
            scatter(NCH - NBUF + i, i).wait()

    out = sc_embed(x3, pos, W)
    return out.reshape(B, L, D)


# speedup vs baseline: 7.9815x; 1.7854x over previous
import functools
import math

import jax
import jax.numpy as jnp
from jax import lax
from jax.experimental import pallas as pl
from jax.experimental.pallas import tpu as pltpu
from jax.experimental.pallas import tpu_sc as plsc

NC = 2
NS = 16
NW = NC * NS
LANES = 16
CHUNK = 100
NBUF = 4
AHEAD = 2


def _pos_table(L, D):
    def body(o_ref):
        j = lax.broadcasted_iota(jnp.int32, (L, D), 1)
        pos = lax.broadcasted_iota(jnp.int32, (L, D), 0).astype(jnp.float32)
        k = (j // 2).astype(jnp.float32)
        div = jnp.exp(k * (-2.0 * math.log(10000.0) / D))
        ang = pos * div
        o_ref[...] = jnp.where(j % 2 == 0, jnp.sin(ang), jnp.cos(ang))

    return pl.pallas_call(
        body, out_shape=jax.ShapeDtypeStruct((L, D), jnp.float32)
    )()


@jax.jit
def kernel(x, W):
    B, L = x.shape
    V, D = W.shape
    N = B * L
    assert N % NW == 0
    PW = N // NW
    assert PW % CHUNK == 0
    NCH = PW // CHUNK
    assert L == 2 * CHUNK
    assert PW % L == 0
    assert NCH % NBUF == 0 and NCH // NBUF >= 3
    DV = D // LANES

    pos = _pos_table(L, D).reshape(2, CHUNK, D)
    x3 = x.astype(jnp.int32).reshape(NW, NCH, CHUNK)

    @functools.partial(
        pl.kernel,
        out_type=jax.ShapeDtypeStruct((N, D), jnp.float32),
        mesh=plsc.VectorSubcoreMesh(core_axis_name="c", subcore_axis_name="s"),
        scratch_types=[
            pltpu.VMEM((NCH, CHUNK), jnp.int32),
            pltpu.VMEM((2, CHUNK, D), jnp.float32),
            pltpu.VMEM((NBUF, CHUNK, D), jnp.float32),
        ]
        + [pltpu.SemaphoreType.DMA] * (2 * NBUF),
        compiler_params=pltpu.CompilerParams(use_tc_tiling_on_sc=False),
    )
    def sc_embed(x_hbm, pos_hbm, W_hbm, out_hbm, idx_v, pos_v, gbuf, *sems):
        gsem = sems[:NBUF]
        ssem = sems[NBUF:]
        wid = lax.axis_index("s") * NC + lax.axis_index("c")
        base = wid * PW
        pltpu.sync_copy(x_hbm.at[wid], idx_v)
        pltpu.sync_copy(pos_hbm, pos_v)

        def gather(ch, b):
            return pltpu.make_async_copy(
                W_hbm.at[idx_v.at[ch]], gbuf.at[b], gsem[b]
            )

        def scatter(ch, b):
            return pltpu.make_async_copy(
                gbuf.at[b], out_hbm.at[pl.ds(base + ch * CHUNK, CHUNK)], ssem[b]
            )

        def add_pos(b, half):
            def row(j, carry):
                for kcol in range(DV):
                    sl = pl.ds(kcol * LANES, LANES)
                    plsc.addupdate(gbuf.at[b, j, sl], pos_v[half, j, sl])
                return carry

            lax.fori_loop(0, CHUNK, row, 0, unroll=2)

        def group(p, first, last):
            for i in range(NBUF):
                c = NBUF * p + i
                gather(c, i).wait()
                add_pos(i, i % 2)
                scatter(c, i).start()
                ca = c + AHEAD
                bb = (i + AHEAD) % NBUF
                if not (last and i >= NBUF - AHEAD):
                    if not (first and i < AHEAD):
                        scatter(ca - NBUF, bb).wait()
                    gather(ca, bb).start()

        for c in range(AHEAD):
            gather(c, c).start()
        group(0, True, False)
        lax.fori_loop(1, NCH // NBUF - 1, lambda p, _: (group(p, False, False), 0)[1], 0)
        group(NCH // NBUF - 1, False, True)
        for i in range(NBUF):
            scatter(NCH - NBUF + i, i).wait()

    out = sc_embed(x3, pos, W)
    return out.reshape(B, L, D)
